# baseline (speedup 1.0000x reference)
import jax
import jax.numpy as jnp
from jax import lax
from jax.experimental import pallas as pl
from jax.experimental.pallas import tpu as pltpu
from jax.experimental.pallas import tpu_sc as plsc

N_NODES = 10000
D_FEAT = 256
N_EDGES = 160000

NC = 2
NS = 16
L = 16
NW = NC * NS
EDGES_PER_WORKER = N_EDGES // NW
CHUNK = 64
NCHUNKS = 80
LAST_BASE = EDGES_PER_WORKER - CHUNK


def _nr_sqrt(x):
    i = plsc.bitcast(x, jnp.int32)
    i = jnp.int32(0x5F3759DF) - (i >> 1)
    y = plsc.bitcast(i, jnp.float32)
    for _ in range(3):
        y = y * (jnp.float32(1.5) - jnp.float32(0.5) * x * y * y)
    return x * y


def _edge_dist_body(h_hbm, src_hbm, dst_hbm, out_hbm,
                    idx_s0, idx_s1, idx_d0, idx_d1,
                    rows_s0, rows_s1, rows_d0, rows_d1,
                    out_v0, out_v1, acc_mat,
                    sem_i0, sem_i1, sem_r0, sem_r1, sem_o0, sem_o1):
    idx_s = (idx_s0, idx_s1)
    idx_d = (idx_d0, idx_d1)
    rows_s = (rows_s0, rows_s1)
    rows_d = (rows_d0, rows_d1)
    out_v = (out_v0, out_v1)
    sem_i = (sem_i0, sem_i1)
    sem_r = (sem_r0, sem_r1)
    sem_o = (sem_o0, sem_o1)

    wid = lax.axis_index("s") * NC + lax.axis_index("c")
    ebase = wid * EDGES_PER_WORKER
    lane = jnp.arange(L, dtype=jnp.int32)

    def base_of(ch):
        return pl.multiple_of(ebase + jnp.minimum(ch * CHUNK, LAST_BASE), 8)

    def start_idx(ch, b):
        base = base_of(ch)
        pltpu.async_copy(src_hbm.at[pl.ds(base, CHUNK)], idx_s[b], sem_i[b])
        pltpu.async_copy(dst_hbm.at[pl.ds(base, CHUNK)], idx_d[b], sem_i[b])

    def wait_idx(b):
        pltpu.make_async_copy(src_hbm.at[pl.ds(0, CHUNK)], idx_s[b], sem_i[b]).wait()
        pltpu.make_async_copy(dst_hbm.at[pl.ds(0, CHUNK)], idx_d[b], sem_i[b]).wait()

    def start_rows(b):
        pltpu.async_copy(h_hbm.at[idx_s[b]], rows_s[b], sem_r[b])
        pltpu.async_copy(h_hbm.at[idx_d[b]], rows_d[b], sem_r[b])

    def wait_rows(b):
        pltpu.make_async_copy(h_hbm.at[idx_s[b]], rows_s[b], sem_r[b]).wait()
        pltpu.make_async_copy(h_hbm.at[idx_d[b]], rows_d[b], sem_r[b]).wait()

    def wait_out(b):
        pltpu.make_async_copy(out_v[b], out_hbm.at[pl.ds(0, CHUNK)], sem_o[b]).wait()

    def compute(ch, b):
        zero = jnp.zeros((L,), jnp.float32)
        for g in range(CHUNK // L):
            @plsc.parallel_loop(0, L, 1, unroll=4)
            def edge_body(r):
                e = jnp.int32(g * L) + r
                halves = []
                for half in range(2):
                    ab = None
                    for jj in range(4):
                        j = half * 4 + jj
                        s = rows_s[b][e, pl.ds(j * 2 * L, 2 * L)]
                        d = rows_d[b][e, pl.ds(j * 2 * L, 2 * L)]
                        df = d - s
                        sq = df * df
                        ab = sq if ab is None else ab + sq
                    lo, hi = plsc.unpack(ab, format=plsc.PackFormat.INTERLEAVED)
                    halves.append(lo + hi)
                acc_mat[r] = halves[0] + halves[1]

            t = [zero, zero, zero, zero]
            for l in range(L):
                col = jnp.full((L,), l, dtype=jnp.int32)
                t[l % 4] = t[l % 4] + plsc.load_gather(acc_mat, [lane, col])
            dist2 = (t[0] + t[1]) + (t[2] + t[3])
            out_v[b][pl.ds(g * L, L)] = _nr_sqrt(dist2)
        pltpu.async_copy(out_v[b], out_hbm.at[pl.ds(base_of(ch), CHUNK)],
                         sem_o[b])

    start_idx(0, 0)
    start_idx(1, 1)
    wait_idx(0)
    start_rows(0)

    def pair_body(i, carry):
        for b in (0, 1):
            ch = 2 * i + b
            nb = 1 - b
            wait_idx(nb)
            wait_rows(b)
            start_rows(nb)
            start_idx(ch + 2, b)

            @pl.when(ch >= 2)
            def _():
                wait_out(b)

            compute(ch, b)
        return carry

    lax.fori_loop(0, NCHUNKS // 2, pair_body, 0, unroll=False)

    wait_rows(0)
    wait_idx(1)
    wait_out(0)
    wait_out(1)


@jax.jit
def kernel(h, edge_index):
    src = edge_index[0].astype(jnp.int32)
    dst = edge_index[1].astype(jnp.int32)
    hb = h.astype(jnp.bfloat16)
    mesh = plsc.VectorSubcoreMesh(core_axis_name="c", subcore_axis_name="s")
    out = pl.kernel(
        _edge_dist_body,
        out_type=jax.ShapeDtypeStruct((N_EDGES,), jnp.float32),
        mesh=mesh,
        compiler_params=pltpu.CompilerParams(use_tc_tiling_on_sc=False,
                                             needs_layout_passes=False),
        scratch_types=[
            pltpu.VMEM((CHUNK,), jnp.int32),
            pltpu.VMEM((CHUNK,), jnp.int32),
            pltpu.VMEM((CHUNK,), jnp.int32),
            pltpu.VMEM((CHUNK,), jnp.int32),
            pltpu.VMEM((CHUNK, D_FEAT), jnp.bfloat16),
            pltpu.VMEM((CHUNK, D_FEAT), jnp.bfloat16),
            pltpu.VMEM((CHUNK, D_FEAT), jnp.bfloat16),
            pltpu.VMEM((CHUNK, D_FEAT), jnp.bfloat16),
            pltpu.VMEM((CHUNK,), jnp.float32),
            pltpu.VMEM((CHUNK,), jnp.float32),
            pltpu.VMEM((L, L), jnp.float32),
            pltpu.SemaphoreType.DMA,
            pltpu.SemaphoreType.DMA,
            pltpu.SemaphoreType.DMA,
            pltpu.SemaphoreType.DMA,
            pltpu.SemaphoreType.DMA,
            pltpu.SemaphoreType.DMA,
        ],
    )(hb, src, dst)
    return out.reshape(N_EDGES, 1)


# speedup vs baseline: 1.0023x; 1.0023x over previous
import jax
import jax.numpy as jnp
from jax import lax
from jax.experimental import pallas as pl
from jax.experimental.pallas import tpu as pltpu
from jax.experimental.pallas import tpu_sc as plsc

N_NODES = 10000
D_FEAT = 256
N_EDGES = 160000

NC = 2
NS = 16
L = 16
NW = NC * NS
EDGES_PER_WORKER = N_EDGES // NW
CHUNK = 64
NCHUNKS = 80
LAST_BASE = EDGES_PER_WORKER - CHUNK


def _nr_sqrt(x):
    i = plsc.bitcast(x, jnp.int32)
    i = jnp.int32(0x5F3759DF) - (i >> 1)
    y = plsc.bitcast(i, jnp.float32)
    for _ in range(3):
        y = y * (jnp.float32(1.5) - jnp.float32(0.5) * x * y * y)
    return x * y


def _edge_dist_body(h_hbm, src_hbm, dst_hbm, out_hbm,
                    idx_s0, idx_s1, idx_d0, idx_d1,
                    rows_s0, rows_s1, rows_d0, rows_d1,
                    out_v0, out_v1, acc_mat,
                    sem_i0, sem_i1, sem_r0, sem_r1, sem_o0, sem_o1):
    idx_s = (idx_s0, idx_s1)
    idx_d = (idx_d0, idx_d1)
    rows_s = (rows_s0, rows_s1)
    rows_d = (rows_d0, rows_d1)
    out_v = (out_v0, out_v1)
    sem_i = (sem_i0, sem_i1)
    sem_r = (sem_r0, sem_r1)
    sem_o = (sem_o0, sem_o1)

    wid = lax.axis_index("s") * NC + lax.axis_index("c")
    ebase = wid * EDGES_PER_WORKER
    lane = jnp.arange(L, dtype=jnp.int32)

    def base_of(ch):
        return pl.multiple_of(ebase + jnp.minimum(ch * CHUNK, LAST_BASE), 8)

    def start_idx(ch, b):
        base = base_of(ch)
        pltpu.async_copy(src_hbm.at[pl.ds(base, CHUNK)], idx_s[b], sem_i[b])
        pltpu.async_copy(dst_hbm.at[pl.ds(base, CHUNK)], idx_d[b], sem_i[b])

    def wait_idx(b):
        pltpu.make_async_copy(src_hbm.at[pl.ds(0, CHUNK)], idx_s[b], sem_i[b]).wait()
        pltpu.make_async_copy(dst_hbm.at[pl.ds(0, CHUNK)], idx_d[b], sem_i[b]).wait()

    def start_rows(b):
        pltpu.async_copy(h_hbm.at[idx_s[b]], rows_s[b], sem_r[b])
        pltpu.async_copy(h_hbm.at[idx_d[b]], rows_d[b], sem_r[b])

    def wait_rows(b):
        pltpu.make_async_copy(h_hbm.at[idx_s[b]], rows_s[b], sem_r[b]).wait()
        pltpu.make_async_copy(h_hbm.at[idx_d[b]], rows_d[b], sem_r[b]).wait()

    def wait_out(b):
        pltpu.make_async_copy(out_v[b], out_hbm.at[pl.ds(0, CHUNK)], sem_o[b]).wait()

    def compute(ch, b):
        zero = jnp.zeros((L,), jnp.float32)
        for g in range(CHUNK // L):
            @plsc.parallel_loop(0, L, 1, unroll=2, carry=zero)
            def dist2(r, d2):
                e = jnp.int32(g * L) + r
                halves = []
                for half in range(2):
                    ab = None
                    for jj in range(4):
                        j = half * 4 + jj
                        s = rows_s[b][e, pl.ds(j * 2 * L, 2 * L)]
                        d = rows_d[b][e, pl.ds(j * 2 * L, 2 * L)]
                        df = d - s
                        sq = df * df
                        ab = sq if ab is None else ab + sq
                    lo, hi = plsc.unpack(ab, format=plsc.PackFormat.INTERLEAVED)
                    halves.append(lo + hi)
                acc = halves[0] + halves[1]
                cs = jnp.cumsum(acc)
                tot = cs.at[jnp.full((L,), L - 1, jnp.int32)].get(
                    mode='promise_in_bounds')
                return jnp.where(lane == r, tot, d2)

            out_v[b][pl.ds(g * L, L)] = _nr_sqrt(dist2)
        pltpu.async_copy(out_v[b], out_hbm.at[pl.ds(base_of(ch), CHUNK)],
                         sem_o[b])

    start_idx(0, 0)
    start_idx(1, 1)
    wait_idx(0)
    start_rows(0)

    def pair_body(i, carry):
        for b in (0, 1):
            ch = 2 * i + b
            nb = 1 - b
            wait_idx(nb)
            wait_rows(b)
            start_rows(nb)
            start_idx(ch + 2, b)

            @pl.when(ch >= 2)
            def _():
                wait_out(b)

            compute(ch, b)
        return carry

    lax.fori_loop(0, NCHUNKS // 2, pair_body, 0, unroll=False)

    wait_rows(0)
    wait_idx(1)
    wait_out(0)
    wait_out(1)


@jax.jit
def kernel(h, edge_index):
    src = edge_index[0].astype(jnp.int32)
    dst = edge_index[1].astype(jnp.int32)
    hb = h.astype(jnp.bfloat16)
    mesh = plsc.VectorSubcoreMesh(core_axis_name="c", subcore_axis_name="s")
    out = pl.kernel(
        _edge_dist_body,
        out_type=jax.ShapeDtypeStruct((N_EDGES,), jnp.float32),
        mesh=mesh,
        compiler_params=pltpu.CompilerParams(use_tc_tiling_on_sc=False,
                                             needs_layout_passes=False),
        scratch_types=[
            pltpu.VMEM((CHUNK,), jnp.int32),
            pltpu.VMEM((CHUNK,), jnp.int32),
            pltpu.VMEM((CHUNK,), jnp.int32),
            pltpu.VMEM((CHUNK,), jnp.int32),
            pltpu.VMEM((CHUNK, D_FEAT), jnp.bfloat16),
            pltpu.VMEM((CHUNK, D_FEAT), jnp.bfloat16),
            pltpu.VMEM((CHUNK, D_FEAT), jnp.bfloat16),
            pltpu.VMEM((CHUNK, D_FEAT), jnp.bfloat16),
            pltpu.VMEM((CHUNK,), jnp.float32),
            pltpu.VMEM((CHUNK,), jnp.float32),
            pltpu.VMEM((L, L), jnp.float32),
            pltpu.SemaphoreType.DMA,
            pltpu.SemaphoreType.DMA,
            pltpu.SemaphoreType.DMA,
            pltpu.SemaphoreType.DMA,
            pltpu.SemaphoreType.DMA,
            pltpu.SemaphoreType.DMA,
        ],
    )(hb, src, dst)
    return out.reshape(N_EDGES, 1)


# speedup vs baseline: 1.3951x; 1.3919x over previous
import jax
import jax.numpy as jnp
from jax import lax
from jax.experimental import pallas as pl
from jax.experimental.pallas import tpu as pltpu
from jax.experimental.pallas import tpu_sc as plsc

N_NODES = 10000
D_FEAT = 256
N_EDGES = 160000

NC = 2
NS = 16
L = 16
NW = NC * NS
EDGES_PER_WORKER = N_EDGES // NW
CHUNK = 64
NCHUNKS = 80
LAST_BASE = EDGES_PER_WORKER - CHUNK


def _nr_sqrt(x):
    i = plsc.bitcast(x, jnp.int32)
    i = jnp.int32(0x5F3759DF) - (i >> 1)
    y = plsc.bitcast(i, jnp.float32)
    for _ in range(3):
        y = y * (jnp.float32(1.5) - jnp.float32(0.5) * x * y * y)
    return x * y


def _edge_dist_body(h_hbm, src_hbm, dst_hbm, out_hbm,
                    h_sp,
                    idx_s0, idx_s1, idx_d0, idx_d1,
                    rows_s0, rows_s1, rows_d0, rows_d1,
                    out_v0, out_v1, acc_mat,
                    sem_i0, sem_i1, sem_r0, sem_r1, sem_o0, sem_o1):
    idx_s = (idx_s0, idx_s1)
    idx_d = (idx_d0, idx_d1)
    rows_s = (rows_s0, rows_s1)
    rows_d = (rows_d0, rows_d1)
    out_v = (out_v0, out_v1)
    sem_i = (sem_i0, sem_i1)
    sem_r = (sem_r0, sem_r1)
    sem_o = (sem_o0, sem_o1)

    wid = lax.axis_index("s") * NC + lax.axis_index("c")
    ebase = wid * EDGES_PER_WORKER
    lane = jnp.arange(L, dtype=jnp.int32)

    def base_of(ch):
        return pl.multiple_of(ebase + jnp.minimum(ch * CHUNK, LAST_BASE), 8)

    def start_idx(ch, b):
        base = base_of(ch)
        pltpu.async_copy(src_hbm.at[pl.ds(base, CHUNK)], idx_s[b], sem_i[b])
        pltpu.async_copy(dst_hbm.at[pl.ds(base, CHUNK)], idx_d[b], sem_i[b])

    def wait_idx(b):
        pltpu.make_async_copy(src_hbm.at[pl.ds(0, CHUNK)], idx_s[b], sem_i[b]).wait()
        pltpu.make_async_copy(dst_hbm.at[pl.ds(0, CHUNK)], idx_d[b], sem_i[b]).wait()

    def start_rows(b):
        pltpu.async_copy(h_sp.at[idx_s[b]], rows_s[b], sem_r[b])
        pltpu.async_copy(h_sp.at[idx_d[b]], rows_d[b], sem_r[b])

    def wait_rows(b):
        pltpu.make_async_copy(h_sp.at[idx_s[b]], rows_s[b], sem_r[b]).wait()
        pltpu.make_async_copy(h_sp.at[idx_d[b]], rows_d[b], sem_r[b]).wait()

    def wait_out(b):
        pltpu.make_async_copy(out_v[b], out_hbm.at[pl.ds(0, CHUNK)], sem_o[b]).wait()

    def compute(ch, b):
        zero = jnp.zeros((L,), jnp.float32)
        for g in range(CHUNK // L):
            @plsc.parallel_loop(0, L, 1, unroll=2, carry=zero)
            def dist2(r, d2):
                e = jnp.int32(g * L) + r
                halves = []
                for half in range(2):
                    ab = None
                    for jj in range(4):
                        j = half * 4 + jj
                        s = rows_s[b][e, pl.ds(j * 2 * L, 2 * L)]
                        d = rows_d[b][e, pl.ds(j * 2 * L, 2 * L)]
                        df = d - s
                        sq = df * df
                        ab = sq if ab is None else ab + sq
                    lo, hi = plsc.unpack(ab, format=plsc.PackFormat.INTERLEAVED)
                    halves.append(lo + hi)
                acc = halves[0] + halves[1]
                cs = jnp.cumsum(acc)
                tot = cs.at[jnp.full((L,), L - 1, jnp.int32)].get(
                    mode='promise_in_bounds')
                return jnp.where(lane == r, tot, d2)

            out_v[b][pl.ds(g * L, L)] = _nr_sqrt(dist2)
        pltpu.async_copy(out_v[b], out_hbm.at[pl.ds(base_of(ch), CHUNK)],
                         sem_o[b])

    sid = lax.axis_index("s")
    rows_per_tile = N_NODES // NS
    pltpu.sync_copy(h_hbm.at[pl.ds(sid * rows_per_tile, rows_per_tile)],
                    h_sp.at[pl.ds(sid * rows_per_tile, rows_per_tile)])
    plsc.subcore_barrier()

    start_idx(0, 0)
    start_idx(1, 1)
    wait_idx(0)
    start_rows(0)

    def pair_body(i, carry):
        for b in (0, 1):
            ch = 2 * i + b
            nb = 1 - b
            wait_idx(nb)
            wait_rows(b)
            start_rows(nb)
            start_idx(ch + 2, b)

            @pl.when(ch >= 2)
            def _():
                wait_out(b)

            compute(ch, b)
        return carry

    lax.fori_loop(0, NCHUNKS // 2, pair_body, 0, unroll=False)

    wait_rows(0)
    wait_idx(1)
    wait_out(0)
    wait_out(1)


@jax.jit
def kernel(h, edge_index):
    src = edge_index[0].astype(jnp.int32)
    dst = edge_index[1].astype(jnp.int32)
    hb = h.astype(jnp.bfloat16)
    mesh = plsc.VectorSubcoreMesh(core_axis_name="c", subcore_axis_name="s")
    out = pl.kernel(
        _edge_dist_body,
        out_type=jax.ShapeDtypeStruct((N_EDGES,), jnp.float32),
        mesh=mesh,
        compiler_params=pltpu.CompilerParams(use_tc_tiling_on_sc=False,
                                             needs_layout_passes=False),
        scratch_types=[
            pltpu.VMEM_SHARED((N_NODES, D_FEAT), jnp.bfloat16),
            pltpu.VMEM((CHUNK,), jnp.int32),
            pltpu.VMEM((CHUNK,), jnp.int32),
            pltpu.VMEM((CHUNK,), jnp.int32),
            pltpu.VMEM((CHUNK,), jnp.int32),
            pltpu.VMEM((CHUNK, D_FEAT), jnp.bfloat16),
            pltpu.VMEM((CHUNK, D_FEAT), jnp.bfloat16),
            pltpu.VMEM((CHUNK, D_FEAT), jnp.bfloat16),
            pltpu.VMEM((CHUNK, D_FEAT), jnp.bfloat16),
            pltpu.VMEM((CHUNK,), jnp.float32),
            pltpu.VMEM((CHUNK,), jnp.float32),
            pltpu.VMEM((L, L), jnp.float32),
            pltpu.SemaphoreType.DMA,
            pltpu.SemaphoreType.DMA,
            pltpu.SemaphoreType.DMA,
            pltpu.SemaphoreType.DMA,
            pltpu.SemaphoreType.DMA,
            pltpu.SemaphoreType.DMA,
        ],
    )(hb, src, dst)
    return out.reshape(N_EDGES, 1)
